# baseline (speedup 1.0000x reference)
import functools

import jax
import jax.numpy as jnp
from jax import lax
from jax.experimental import pallas as pl
from jax.experimental.pallas import tpu as pltpu
from jax.experimental.pallas import tpu_sc as plsc

_QT = 512


def _bf16_rne(x):
    bits = lax.bitcast_convert_type(x, jnp.uint32)
    lsb = (bits >> 16) & jnp.uint32(1)
    rounded = (bits + jnp.uint32(0x7FFF) + lsb) & jnp.uint32(0xFFFF0000)
    return lax.bitcast_convert_type(rounded, jnp.float32)


def _argmin_body(n_keys, pred_ref, realt_ref, iota_ref, out_ref):
    qp = pred_ref[0]
    qx = qp[:, 0:1]
    qy = qp[:, 1:2]
    r = realt_ref[0]
    kx = r[0:1, :]
    ky = r[1:2, :]
    q2 = qx * qx + qy * qy
    r2 = kx * kx + ky * ky
    cross2 = lax.dot_general(
        qp.astype(jnp.bfloat16), (r * 2.0).astype(jnp.bfloat16),
        (((1,), (0,)), ((), ())),
        preferred_element_type=jnp.float32)
    d2 = (q2 + r2) - cross2
    md2 = jnp.min(d2, axis=1)
    md2c = jnp.maximum(md2, 0.0).reshape(1, md2.shape[0])
    m = jnp.sqrt(md2c)
    ub = lax.bitcast_convert_type(md2c, jnp.uint32)
    u = md2c
    for k in (1, 2, 3, 4):
        xk = lax.bitcast_convert_type(ub + jnp.uint32(k), jnp.float32)
        u = jnp.where(jnp.sqrt(xk) == m, xk, u)
    u = u.reshape(md2.shape[0], 1)
    ii = iota_ref[0]
    cand = jnp.where(d2 <= u, ii, jnp.float32(n_keys))
    idx = jnp.min(cand, axis=1)
    b = pl.program_id(0)
    out_ref[0, 0, 0, :] = idx.astype(jnp.int32) + b * n_keys


def _nearest_indices(pred, realt):
    b, n, p = pred.shape
    nq = n // _QT
    iota = jnp.arange(n, dtype=jnp.float32).reshape(1, n)
    out = pl.pallas_call(
        functools.partial(_argmin_body, n),
        grid=(b, nq),
        in_specs=[
            pl.BlockSpec((1, _QT, p), lambda bi, qi: (bi, qi, 0)),
            pl.BlockSpec((1, p, n), lambda bi, qi: (bi, 0, 0)),
            pl.BlockSpec((1, n), lambda bi, qi: (0, 0)),
        ],
        out_specs=pl.BlockSpec((1, 1, 1, _QT), lambda bi, qi: (bi, qi, 0, 0)),
        out_shape=jax.ShapeDtypeStruct((b, nq, 1, _QT), jnp.int32),
    )(pred, realt, iota)
    return out.reshape(b * n)


def _make_sc_gather(bn, g):
    info = plsc.get_sparse_core_info()
    nc, ns = info.num_cores, info.num_subcores
    nw = nc * ns
    rpw = bn // nw
    ch = 64
    nch = rpw // ch

    mesh = plsc.VectorSubcoreMesh(core_axis_name="c", subcore_axis_name="s")

    @functools.partial(
        pl.kernel,
        mesh=mesh,
        out_type=jax.ShapeDtypeStruct((bn, g), jnp.float32),
        scratch_types=[
            pltpu.VMEM((nch, ch), jnp.int32),
            pltpu.VMEM((ch, g), jnp.float32),
            pltpu.VMEM((ch, g), jnp.float32),
            pltpu.VMEM((ch, g), jnp.float32),
            pltpu.SemaphoreType.DMA,
            pltpu.SemaphoreType.DMA,
            pltpu.SemaphoreType.DMA,
            pltpu.SemaphoreType.DMA,
            pltpu.SemaphoreType.DMA,
            pltpu.SemaphoreType.DMA,
        ],
    )
    def gather(table_hbm, idx_hbm, out_hbm, idx_v, buf0, buf1, buf2,
               gs0, gs1, gs2, os0, os1, os2):
        wid = lax.axis_index("s") * nc + lax.axis_index("c")
        base = wid * rpw
        pltpu.sync_copy(idx_hbm.at[wid], idx_v)
        bufs = (buf0, buf1, buf2)
        gsems = (gs0, gs1, gs2)
        osems = (os0, os1, os2)
        gh = {}
        oh = {}
        for c in (0, 1):
            gh[c] = pltpu.async_copy(
                table_hbm.at[idx_v.at[c]], bufs[c], gsems[c])
        for c in range(nch):
            nx = c + 2
            if nx < nch:
                if nx - 3 >= 0:
                    oh.pop(nx % 3).wait()
                gh[nx % 3] = pltpu.async_copy(
                    table_hbm.at[idx_v.at[nx]], bufs[nx % 3], gsems[nx % 3])
            gh.pop(c % 3).wait()
            oh[c % 3] = pltpu.async_copy(
                bufs[c % 3], out_hbm.at[pl.ds(base + c * ch, ch)],
                osems[c % 3])
        for h in oh.values():
            h.wait()

    def run(table, idx_flat):
        idx3 = idx_flat.reshape(nw, nch, ch)
        return gather(table, idx3)

    return run


def kernel(predicted_positions, real_positions, real_expressions):
    b, n, p = predicted_positions.shape
    g = real_expressions.shape[-1]
    realt = jnp.swapaxes(real_positions, 1, 2)
    idx_flat = _nearest_indices(predicted_positions, realt)
    table = real_expressions.reshape(b * n, g)
    out = _make_sc_gather(b * n, g)(table, idx_flat)
    return out.reshape(b, n, g)


# speedup vs baseline: 1.0009x; 1.0009x over previous
import functools

import jax
import jax.numpy as jnp
from jax import lax
from jax.experimental import pallas as pl
from jax.experimental.pallas import tpu as pltpu
from jax.experimental.pallas import tpu_sc as plsc

_QT = 512


def _bf16_rne(x):
    bits = lax.bitcast_convert_type(x, jnp.uint32)
    lsb = (bits >> 16) & jnp.uint32(1)
    rounded = (bits + jnp.uint32(0x7FFF) + lsb) & jnp.uint32(0xFFFF0000)
    return lax.bitcast_convert_type(rounded, jnp.float32)


def _argmin_body(n_keys, pred_ref, realt_ref, iota_ref, out_ref):
    qp = pred_ref[0]
    qx = qp[:, 0:1]
    qy = qp[:, 1:2]
    r = realt_ref[0]
    kx = r[0:1, :]
    ky = r[1:2, :]
    q2 = qx * qx + qy * qy
    r2 = kx * kx + ky * ky
    cross2 = lax.dot_general(
        qp.astype(jnp.bfloat16), (r * 2.0).astype(jnp.bfloat16),
        (((1,), (0,)), ((), ())),
        preferred_element_type=jnp.float32)
    d2 = (q2 + r2) - cross2
    md2 = jnp.min(d2, axis=1)
    md2c = jnp.maximum(md2, 0.0).reshape(1, md2.shape[0])
    m = jnp.sqrt(md2c)
    ub = lax.bitcast_convert_type(md2c, jnp.uint32)
    u = md2c
    for k in (1, 2, 3, 4):
        xk = lax.bitcast_convert_type(ub + jnp.uint32(k), jnp.float32)
        u = jnp.where(jnp.sqrt(xk) == m, xk, u)
    u = u.reshape(md2.shape[0], 1)
    ii = iota_ref[0]
    cand = jnp.where(d2 <= u, ii, jnp.float32(n_keys))
    idx = jnp.min(cand, axis=1)
    b = pl.program_id(0)
    out_ref[0, 0, 0, :] = idx.astype(jnp.int32) + b * n_keys


def _nearest_indices(pred, realt):
    b, n, p = pred.shape
    nq = n // _QT
    iota = jnp.arange(n, dtype=jnp.float32).reshape(1, n)
    out = pl.pallas_call(
        functools.partial(_argmin_body, n),
        grid=(b, nq),
        in_specs=[
            pl.BlockSpec((1, _QT, p), lambda bi, qi: (bi, qi, 0)),
            pl.BlockSpec((1, p, n), lambda bi, qi: (bi, 0, 0)),
            pl.BlockSpec((1, n), lambda bi, qi: (0, 0)),
        ],
        out_specs=pl.BlockSpec((1, 1, 1, _QT), lambda bi, qi: (bi, qi, 0, 0)),
        out_shape=jax.ShapeDtypeStruct((b, nq, 1, _QT), jnp.int32),
        compiler_params=pltpu.CompilerParams(
            dimension_semantics=("parallel", "parallel")),
    )(pred, realt, iota)
    return out.reshape(b * n)


def _make_sc_gather(bn, g):
    info = plsc.get_sparse_core_info()
    nc, ns = info.num_cores, info.num_subcores
    nw = nc * ns
    rpw = bn // nw
    ch = 64
    nch = rpw // ch

    mesh = plsc.VectorSubcoreMesh(core_axis_name="c", subcore_axis_name="s")

    @functools.partial(
        pl.kernel,
        mesh=mesh,
        out_type=jax.ShapeDtypeStruct((bn, g), jnp.float32),
        scratch_types=[
            pltpu.VMEM((nch, ch), jnp.int32),
            pltpu.VMEM((ch, g), jnp.float32),
            pltpu.VMEM((ch, g), jnp.float32),
            pltpu.VMEM((ch, g), jnp.float32),
            pltpu.SemaphoreType.DMA,
            pltpu.SemaphoreType.DMA,
            pltpu.SemaphoreType.DMA,
            pltpu.SemaphoreType.DMA,
            pltpu.SemaphoreType.DMA,
            pltpu.SemaphoreType.DMA,
        ],
    )
    def gather(table_hbm, idx_hbm, out_hbm, idx_v, buf0, buf1, buf2,
               gs0, gs1, gs2, os0, os1, os2):
        wid = lax.axis_index("s") * nc + lax.axis_index("c")
        base = wid * rpw
        pltpu.sync_copy(idx_hbm.at[wid], idx_v)
        bufs = (buf0, buf1, buf2)
        gsems = (gs0, gs1, gs2)
        osems = (os0, os1, os2)
        gh = {}
        oh = {}
        for c in (0, 1):
            gh[c] = pltpu.async_copy(
                table_hbm.at[idx_v.at[c]], bufs[c], gsems[c])
        for c in range(nch):
            nx = c + 2
            if nx < nch:
                if nx - 3 >= 0:
                    oh.pop(nx % 3).wait()
                gh[nx % 3] = pltpu.async_copy(
                    table_hbm.at[idx_v.at[nx]], bufs[nx % 3], gsems[nx % 3])
            gh.pop(c % 3).wait()
            oh[c % 3] = pltpu.async_copy(
                bufs[c % 3], out_hbm.at[pl.ds(base + c * ch, ch)],
                osems[c % 3])
        for h in oh.values():
            h.wait()

    def run(table, idx_flat):
        idx3 = idx_flat.reshape(nw, nch, ch)
        return gather(table, idx3)

    return run


def kernel(predicted_positions, real_positions, real_expressions):
    b, n, p = predicted_positions.shape
    g = real_expressions.shape[-1]
    realt = jnp.swapaxes(real_positions, 1, 2)
    idx_flat = _nearest_indices(predicted_positions, realt)
    table = real_expressions.reshape(b * n, g)
    out = _make_sc_gather(b * n, g)(table, idx_flat)
    return out.reshape(b, n, g)


# speedup vs baseline: 1.0022x; 1.0013x over previous
import functools

import jax
import jax.numpy as jnp
from jax import lax
from jax.experimental import pallas as pl
from jax.experimental.pallas import tpu as pltpu
from jax.experimental.pallas import tpu_sc as plsc

_QT = 512


def _argmin_body(n_keys, pred_ref, realt_ref, iota_ref, out_ref):
    qp = pred_ref[0]
    qx = qp[:, 0:1]
    qy = qp[:, 1:2]
    r = realt_ref[0]
    kx = r[0:1, :]
    ky = r[1:2, :]
    q2 = qx * qx + qy * qy
    r2 = kx * kx + ky * ky
    cross2 = lax.dot_general(
        qp.astype(jnp.bfloat16), (r * 2.0).astype(jnp.bfloat16),
        (((1,), (0,)), ((), ())),
        preferred_element_type=jnp.float32)
    d2 = (q2 + r2) - cross2
    md2 = jnp.min(d2, axis=1)
    md2c = jnp.maximum(md2, 0.0).reshape(1, md2.shape[0])
    m = jnp.sqrt(md2c)
    ub = lax.bitcast_convert_type(md2c, jnp.uint32)
    u = md2c
    for k in (1, 2, 3, 4):
        xk = lax.bitcast_convert_type(ub + jnp.uint32(k), jnp.float32)
        u = jnp.where(jnp.sqrt(xk) == m, xk, u)
    u = u.reshape(md2.shape[0], 1)
    ii = iota_ref[0]
    cand = jnp.where(d2 <= u, ii, jnp.float32(n_keys))
    idx = jnp.min(cand, axis=1)
    b = pl.program_id(0)
    out_ref[0, 0, 0, :] = idx.astype(jnp.int32) + b * n_keys


def _nearest_indices(pred, realt):
    b, n, p = pred.shape
    nq = n // _QT
    iota = jnp.arange(n, dtype=jnp.float32).reshape(1, n)
    out = pl.pallas_call(
        functools.partial(_argmin_body, n),
        grid=(b, nq),
        in_specs=[
            pl.BlockSpec((1, _QT, p), lambda bi, qi: (bi, qi, 0)),
            pl.BlockSpec((1, p, n), lambda bi, qi: (bi, 0, 0)),
            pl.BlockSpec((1, n), lambda bi, qi: (0, 0)),
        ],
        out_specs=pl.BlockSpec((1, 1, 1, _QT), lambda bi, qi: (bi, qi, 0, 0)),
        out_shape=jax.ShapeDtypeStruct((b, nq, 1, _QT), jnp.int32),
        compiler_params=pltpu.CompilerParams(
            dimension_semantics=("parallel", "parallel")),
    )(pred, realt, iota)
    return out.reshape(b * n)


def _make_sc_gather(bn, g):
    info = plsc.get_sparse_core_info()
    nc, ns = info.num_cores, info.num_subcores
    nw = nc * ns
    rpw = bn // nw
    ch = 64
    nch = rpw // ch

    mesh = plsc.VectorSubcoreMesh(core_axis_name="c", subcore_axis_name="s")

    @functools.partial(
        pl.kernel,
        mesh=mesh,
        out_type=jax.ShapeDtypeStruct((bn, g), jnp.float32),
        scratch_types=[
            pltpu.VMEM((nch, ch), jnp.int32),
            pltpu.VMEM((ch, g), jnp.float32),
            pltpu.VMEM((ch, g), jnp.float32),
            pltpu.VMEM((ch, g), jnp.float32),
            pltpu.SemaphoreType.DMA,
            pltpu.SemaphoreType.DMA,
            pltpu.SemaphoreType.DMA,
            pltpu.SemaphoreType.DMA,
            pltpu.SemaphoreType.DMA,
            pltpu.SemaphoreType.DMA,
        ],
    )
    def gather(table_hbm, idx_hbm, out_hbm, idx_v, buf0, buf1, buf2,
               gs0, gs1, gs2, os0, os1, os2):
        wid = lax.axis_index("s") * nc + lax.axis_index("c")
        base = wid * rpw
        pltpu.sync_copy(idx_hbm.at[wid], idx_v)
        bufs = (buf0, buf1, buf2)
        gsems = (gs0, gs1, gs2)
        osems = (os0, os1, os2)
        gh = {}
        oh = {}
        for c in (0, 1):
            gh[c] = pltpu.async_copy(
                table_hbm.at[idx_v.at[c]], bufs[c], gsems[c])
        for c in range(nch):
            nx = c + 2
            if nx < nch:
                if nx - 3 >= 0:
                    oh.pop(nx % 3).wait()
                gh[nx % 3] = pltpu.async_copy(
                    table_hbm.at[idx_v.at[nx]], bufs[nx % 3], gsems[nx % 3])
            gh.pop(c % 3).wait()
            oh[c % 3] = pltpu.async_copy(
                bufs[c % 3], out_hbm.at[pl.ds(base + c * ch, ch)],
                osems[c % 3])
        for h in oh.values():
            h.wait()

    def run(table, idx_flat):
        idx3 = idx_flat.reshape(nw, nch, ch)
        return gather(table, idx3)

    return run


def kernel(predicted_positions, real_positions, real_expressions):
    b, n, p = predicted_positions.shape
    g = real_expressions.shape[-1]
    realt = jnp.swapaxes(real_positions, 1, 2)
    idx_flat = _nearest_indices(predicted_positions, realt)
    table = real_expressions.reshape(b * n, g)
    out = _make_sc_gather(b * n, g)(table, idx_flat)
    return out.reshape(b, n, g)
